# baseline (speedup 1.0000x reference)
import math

import jax
import jax.numpy as jnp
from jax import lax
from jax.experimental import pallas as pl
from jax.experimental.pallas import tpu as pltpu
from jax.experimental.pallas import tpu_sc as plsc

VOCAB = 100000
D_MODEL = 1024
SCALE = math.sqrt(D_MODEL)

NC = 2
NS = 16
NW = NC * NS
LANES = 16

B_TOTAL = 4 * 4096
B_PER_W = B_TOTAL // NW
CH = 16
N_CH = B_PER_W // CH
NBUF = 4
N_ROUNDS = N_CH // NBUF


def _emb_kernel(idx_hbm, table_hbm, out_hbm, idx_v, rows0, rows1, rows2,
                rows3, gsem0, gsem1, gsem2, gsem3, ssem0, ssem1, ssem2,
                ssem3):
    wid = lax.axis_index("s") * NC + lax.axis_index("c")
    base = wid * B_PER_W
    rows = (rows0, rows1, rows2, rows3)
    gsem = (gsem0, gsem1, gsem2, gsem3)
    ssem = (ssem0, ssem1, ssem2, ssem3)

    pltpu.sync_copy(idx_hbm.at[pl.ds(base, B_PER_W)], idx_v)

    def gather_desc(b, ci):
        return pltpu.make_async_copy(
            table_hbm.at[idx_v.at[pl.ds(ci * CH, CH)]], rows[b], gsem[b]
        )

    def scatter_desc(b, ci):
        return pltpu.make_async_copy(
            rows[b], out_hbm.at[pl.ds(base + ci * CH, CH)], ssem[b]
        )

    def scale_buf(b):
        def scale_row(r, _):
            for j in range(D_MODEL // LANES):
                col = j * LANES
                rows[b][r, pl.ds(col, LANES)] = (
                    rows[b][r, pl.ds(col, LANES)] * SCALE
                )
            return 0

        lax.fori_loop(0, CH, scale_row, 0)

    for b in range(NBUF):
        gather_desc(b, b).start()

    def round_body(r, _):
        for b in range(NBUF):
            ci = r * NBUF + b
            gather_desc(b, ci).wait()
            scale_buf(b)
            scatter_desc(b, ci).start()
        for b in range(NBUF):
            ci = r * NBUF + b

            @pl.when(r < N_ROUNDS - 1)
            def _():
                scatter_desc(b, ci).wait()
                gather_desc(b, ci + NBUF).start()

        return 0

    lax.fori_loop(0, N_ROUNDS, round_body, 0)

    for b in range(NBUF):
        scatter_desc(b, (N_ROUNDS - 1) * NBUF + b).wait()


@jax.jit
def kernel(input, table):
    idx = input.reshape(-1).astype(jnp.int32)
    mesh = plsc.VectorSubcoreMesh(core_axis_name="c", subcore_axis_name="s")
    out = pl.kernel(
        _emb_kernel,
        out_type=jax.ShapeDtypeStruct((B_TOTAL, D_MODEL), jnp.float32),
        mesh=mesh,
        scratch_types=(
            [pltpu.VMEM((B_PER_W,), jnp.int32)]
            + [pltpu.VMEM((CH, D_MODEL), jnp.float32)] * NBUF
            + [pltpu.SemaphoreType.DMA] * (2 * NBUF)
        ),
    )(idx, table)
    return out.reshape(input.shape + (D_MODEL,))


# speedup vs baseline: 1.0406x; 1.0406x over previous
import math

import jax
import jax.numpy as jnp
from jax import lax
from jax.experimental import pallas as pl
from jax.experimental.pallas import tpu as pltpu
from jax.experimental.pallas import tpu_sc as plsc

VOCAB = 100000
D_MODEL = 1024
SCALE = math.sqrt(D_MODEL)

NC = 2
NS = 16
NW = NC * NS
LANES = 16

B_TOTAL = 4 * 4096
B_PER_W = B_TOTAL // NW
CH = 32
N_CH = B_PER_W // CH
NBUF = 2
N_ROUNDS = N_CH // NBUF


def _emb_kernel(idx_hbm, table_hbm, out_hbm, idx_v, rows0, rows1,
                gsem0, gsem1, ssem0, ssem1):
    wid = lax.axis_index("s") * NC + lax.axis_index("c")
    base = wid * B_PER_W
    rows = (rows0, rows1)
    gsem = (gsem0, gsem1)
    ssem = (ssem0, ssem1)

    pltpu.sync_copy(idx_hbm.at[pl.ds(base, B_PER_W)], idx_v)

    def gather_desc(b, ci):
        return pltpu.make_async_copy(
            table_hbm.at[idx_v.at[pl.ds(ci * CH, CH)]], rows[b], gsem[b]
        )

    def scatter_desc(b, ci):
        return pltpu.make_async_copy(
            rows[b], out_hbm.at[pl.ds(base + ci * CH, CH)], ssem[b]
        )

    def scale_buf(b):
        def scale_row(r, _):
            for j in range(D_MODEL // LANES):
                col = j * LANES
                rows[b][r, pl.ds(col, LANES)] = (
                    rows[b][r, pl.ds(col, LANES)] * SCALE
                )
            return 0

        lax.fori_loop(0, CH, scale_row, 0)

    for b in range(NBUF):
        gather_desc(b, b).start()

    def round_body(r, _):
        for b in range(NBUF):
            ci = r * NBUF + b
            gather_desc(b, ci).wait()
            scale_buf(b)
            scatter_desc(b, ci).start()
        for b in range(NBUF):
            ci = r * NBUF + b

            @pl.when(r < N_ROUNDS - 1)
            def _():
                scatter_desc(b, ci).wait()
                gather_desc(b, ci + NBUF).start()

        return 0

    lax.fori_loop(0, N_ROUNDS, round_body, 0)

    for b in range(NBUF):
        scatter_desc(b, (N_ROUNDS - 1) * NBUF + b).wait()


@jax.jit
def kernel(input, table):
    idx = input.reshape(-1).astype(jnp.int32)
    mesh = plsc.VectorSubcoreMesh(core_axis_name="c", subcore_axis_name="s")
    out = pl.kernel(
        _emb_kernel,
        out_type=jax.ShapeDtypeStruct((B_TOTAL, D_MODEL), jnp.float32),
        mesh=mesh,
        scratch_types=(
            [pltpu.VMEM((B_PER_W,), jnp.int32)]
            + [pltpu.VMEM((CH, D_MODEL), jnp.float32)] * NBUF
            + [pltpu.SemaphoreType.DMA] * (2 * NBUF)
        ),
    )(idx, table)
    return out.reshape(input.shape + (D_MODEL,))


# speedup vs baseline: 1.2123x; 1.1649x over previous
import math

import jax
import jax.numpy as jnp
from jax import lax
from jax.experimental import pallas as pl
from jax.experimental.pallas import tpu as pltpu
from jax.experimental.pallas import tpu_sc as plsc

VOCAB = 100000
D_MODEL = 1024
SCALE = math.sqrt(D_MODEL)

NC = 2
NS = 16
NW = NC * NS
LANES = 16

B_TOTAL = 4 * 4096
B_PER_W = B_TOTAL // NW
CH = 32
N_CH = B_PER_W // CH
NBUF = 3
N_ROUNDS = N_CH // NBUF


def _emb_kernel(idx_hbm, table_hbm, out_hbm, idx_v, rows0, rows1, rows2,
                gsem0, gsem1, gsem2, ssem0, ssem1, ssem2):
    wid = lax.axis_index("s") * NC + lax.axis_index("c")
    base = wid * B_PER_W
    rows = (rows0, rows1, rows2)
    gsem = (gsem0, gsem1, gsem2)
    ssem = (ssem0, ssem1, ssem2)

    pltpu.sync_copy(idx_hbm.at[pl.ds(base, B_PER_W)], idx_v)

    def gather_desc(b, ci):
        return pltpu.make_async_copy(
            table_hbm.at[idx_v.at[pl.ds(ci * CH, CH)]], rows[b], gsem[b]
        )

    def scatter_desc(b, ci):
        return pltpu.make_async_copy(
            rows[b], out_hbm.at[pl.ds(base + ci * CH, CH)], ssem[b]
        )

    def scale_buf(b):
        def scale_row(r, _):
            for j in range(D_MODEL // LANES):
                col = j * LANES
                rows[b][r, pl.ds(col, LANES)] = (
                    rows[b][r, pl.ds(col, LANES)] * SCALE
                )
            return 0

        lax.fori_loop(0, CH, scale_row, 0)

    gather_desc(0, 0).start()
    gather_desc(1, 1).start()

    def step(ci_base, k, refill=True):
        ci = ci_base + k
        b = k
        gather_desc(b, ci).wait()
        scale_buf(b)
        scatter_desc(b, ci).start()

        if not refill:
            return
        nb = (k + 2) % NBUF

        @pl.when(ci == 0)
        def _():
            gather_desc(nb, ci + 2).start()

        @pl.when(jnp.logical_and(ci >= 1, ci + 2 < N_CH))
        def _():
            scatter_desc(nb, ci - 1).wait()
            gather_desc(nb, ci + 2).start()

    def round_body(r, _):
        for k in range(NBUF):
            step(r * NBUF, k)
        return 0

    lax.fori_loop(0, N_CH // NBUF, round_body, 0)
    step((N_CH // NBUF) * NBUF, 0, refill=False)

    scatter_desc(1, N_CH - 3).wait()
    scatter_desc(2, N_CH - 2).wait()
    scatter_desc(0, N_CH - 1).wait()


@jax.jit
def kernel(input, table):
    idx = input.reshape(-1).astype(jnp.int32)
    mesh = plsc.VectorSubcoreMesh(core_axis_name="c", subcore_axis_name="s")
    out = pl.kernel(
        _emb_kernel,
        out_type=jax.ShapeDtypeStruct((B_TOTAL, D_MODEL), jnp.float32),
        mesh=mesh,
        scratch_types=(
            [pltpu.VMEM((B_PER_W,), jnp.int32)]
            + [pltpu.VMEM((CH, D_MODEL), jnp.float32)] * NBUF
            + [pltpu.SemaphoreType.DMA] * (2 * NBUF)
        ),
    )(idx, table)
    return out.reshape(input.shape + (D_MODEL,))


# speedup vs baseline: 1.2135x; 1.0010x over previous
import math

import jax
import jax.numpy as jnp
from jax import lax
from jax.experimental import pallas as pl
from jax.experimental.pallas import tpu as pltpu
from jax.experimental.pallas import tpu_sc as plsc

VOCAB = 100000
D_MODEL = 1024
SCALE = math.sqrt(D_MODEL)

NC = 2
NS = 16
NW = NC * NS
LANES = 16

B_TOTAL = 4 * 4096
B_PER_W = B_TOTAL // NW
CH = 32
N_CH = B_PER_W // CH
NBUF = 3
N_ROUNDS = N_CH // NBUF


W_PER_G = 4096 // B_PER_W


def _emb_kernel(idx_hbm, table_hbm, out_hbm, idx_v, rows0, rows1, rows2,
                gsem0, gsem1, gsem2, ssem0, ssem1, ssem2):
    wid = lax.axis_index("s") * NC + lax.axis_index("c")
    g = wid // W_PER_G
    base = (wid % W_PER_G) * B_PER_W
    rows = (rows0, rows1, rows2)
    gsem = (gsem0, gsem1, gsem2)
    ssem = (ssem0, ssem1, ssem2)

    pltpu.sync_copy(idx_hbm.at[g, pl.ds(base, B_PER_W)], idx_v)

    def gather_desc(b, ci):
        return pltpu.make_async_copy(
            table_hbm.at[idx_v.at[pl.ds(ci * CH, CH)]], rows[b], gsem[b]
        )

    def scatter_desc(b, ci):
        return pltpu.make_async_copy(
            rows[b], out_hbm.at[g, pl.ds(base + ci * CH, CH)], ssem[b]
        )

    def scale_buf(b):
        def scale_row(r, _):
            for j in range(D_MODEL // LANES):
                col = j * LANES
                rows[b][r, pl.ds(col, LANES)] = (
                    rows[b][r, pl.ds(col, LANES)] * SCALE
                )
            return 0

        lax.fori_loop(0, CH, scale_row, 0)

    gather_desc(0, 0).start()
    gather_desc(1, 1).start()

    def step(ci_base, k, refill=True):
        ci = ci_base + k
        b = k
        gather_desc(b, ci).wait()
        scale_buf(b)
        scatter_desc(b, ci).start()

        if not refill:
            return
        nb = (k + 2) % NBUF

        @pl.when(ci == 0)
        def _():
            gather_desc(nb, ci + 2).start()

        @pl.when(jnp.logical_and(ci >= 1, ci + 2 < N_CH))
        def _():
            scatter_desc(nb, ci - 1).wait()
            gather_desc(nb, ci + 2).start()

    def round_body(r, _):
        for k in range(NBUF):
            step(r * NBUF, k)
        return 0

    lax.fori_loop(0, N_CH // NBUF, round_body, 0)
    step((N_CH // NBUF) * NBUF, 0, refill=False)

    scatter_desc(1, N_CH - 3).wait()
    scatter_desc(2, N_CH - 2).wait()
    scatter_desc(0, N_CH - 1).wait()


@jax.jit
def kernel(input, table):
    idx = input.astype(jnp.int32)
    mesh = plsc.VectorSubcoreMesh(core_axis_name="c", subcore_axis_name="s")
    return pl.kernel(
        _emb_kernel,
        out_type=jax.ShapeDtypeStruct(input.shape + (D_MODEL,), jnp.float32),
        mesh=mesh,
        scratch_types=(
            [pltpu.VMEM((B_PER_W,), jnp.int32)]
            + [pltpu.VMEM((CH, D_MODEL), jnp.float32)] * NBUF
            + [pltpu.SemaphoreType.DMA] * (2 * NBUF)
        ),
    )(idx, table)
